# baseline (speedup 1.0000x reference)
import jax
import jax.numpy as jnp
from jax import lax
from jax.experimental import pallas as pl
from jax.experimental.pallas import tpu as pltpu
from jax.experimental.pallas import tpu_sc as plsc

N = 100000
B = 4096
K = 16
D = 128
DE = 16
DT = 100
H = 2
DH = D // H
E = B + B * K

_NC = 2
_NS = 16
_NW = _NC * _NS
_CH = 128
_PER_W = E // _NW
_NCHUNK = _PER_W // _CH


def _sc_gather_mn_body(nodes, mem, nfeat, o_mem, o_nfeat,
                       idx_all, mem_v, nfeat_v, sem):
    cid = lax.axis_index("c")
    sid = lax.axis_index("s")
    wid = sid * _NC + cid
    base = wid * _PER_W
    pltpu.sync_copy(nodes.at[pl.ds(base, _PER_W)], idx_all)

    def chunk(c, carry):
        off = base + c * _CH
        idx = idx_all.at[pl.ds(c * _CH, _CH)]
        cp1 = pltpu.async_copy(mem.at[idx], mem_v, sem)
        cp2 = pltpu.async_copy(nfeat.at[idx], nfeat_v, sem)
        cp1.wait()
        cp2.wait()
        pltpu.sync_copy(mem_v, o_mem.at[pl.ds(off, _CH)])
        pltpu.sync_copy(nfeat_v, o_nfeat.at[pl.ds(off, _CH)])
        return carry

    lax.fori_loop(0, _NCHUNK, chunk, 0)


def _sc_gather_mb_body(nodes, mbox, o_mbox, idx_all, mbox_v, sem):
    cid = lax.axis_index("c")
    sid = lax.axis_index("s")
    wid = sid * _NC + cid
    base = wid * _PER_W
    pltpu.sync_copy(nodes.at[pl.ds(base, _PER_W)], idx_all)

    def chunk(c, carry):
        off = base + c * _CH
        idx = idx_all.at[pl.ds(c * _CH, _CH)]
        pltpu.async_copy(mbox.at[idx, pl.ds(0, 2 * D)], mbox_v, sem).wait()
        pltpu.sync_copy(mbox_v, o_mbox.at[pl.ds(off, _CH)])
        return carry

    lax.fori_loop(0, _NCHUNK, chunk, 0)


def _sc_gather_ax_body(nodes, aux, o_aux, idx_all, aux_v, sem):
    cid = lax.axis_index("c")
    sid = lax.axis_index("s")
    wid = sid * _NC + cid
    base = wid * _PER_W
    pltpu.sync_copy(nodes.at[pl.ds(base, _PER_W)], idx_all)

    def chunk(c, carry):
        off = base + c * _CH
        idx = idx_all.at[pl.ds(c * _CH, _CH)]
        pltpu.async_copy(aux.at[idx], aux_v, sem).wait()
        pltpu.sync_copy(aux_v, o_aux.at[pl.ds(off, _CH)])
        return carry

    lax.fori_loop(0, _NCHUNK, chunk, 0)


def _sc_mesh():
    return plsc.VectorSubcoreMesh(core_axis_name="c", subcore_axis_name="s",
                                  num_cores=_NC)


def _gather_mn(nodes, mem, nfeat):
    f32 = jnp.float32
    run = pl.kernel(
        _sc_gather_mn_body,
        mesh=_sc_mesh(),
        out_type=[
            jax.ShapeDtypeStruct((E, D), f32),
            jax.ShapeDtypeStruct((E, D), f32),
        ],
        scratch_types=[
            pltpu.VMEM((_PER_W,), jnp.int32),
            pltpu.VMEM((_CH, D), f32),
            pltpu.VMEM((_CH, D), f32),
            pltpu.SemaphoreType.DMA,
        ],
    )
    return run(nodes, mem, nfeat)


def _gather_mb(nodes, mailbox):
    run = pl.kernel(
        _sc_gather_mb_body,
        mesh=_sc_mesh(),
        out_type=jax.ShapeDtypeStruct((E, 2 * D), jnp.float32),
        scratch_types=[
            pltpu.VMEM((_PER_W,), jnp.int32),
            pltpu.VMEM((_CH, 2 * D), jnp.float32),
            pltpu.SemaphoreType.DMA,
        ],
    )
    return run(nodes, mailbox)


def _gather_ax(nodes, aux):
    run = pl.kernel(
        _sc_gather_ax_body,
        mesh=_sc_mesh(),
        out_type=jax.ShapeDtypeStruct((E, D), jnp.float32),
        scratch_types=[
            pltpu.VMEM((_PER_W,), jnp.int32),
            pltpu.VMEM((_CH, D), jnp.float32),
            pltpu.SemaphoreType.DMA,
        ],
    )
    return run(nodes, aux)


_RAUX = 2000


def _aux_body(tail_ref, mt_ref, met_ref, aux_ref):
    R = _RAUX
    delta_row = (mt_ref[...] - met_ref[...]).reshape(1, R)
    delta_col = delta_row.T
    aux_ref[...] = jnp.concatenate(
        [tail_ref[:, :DE], delta_col, jnp.zeros((R, D - DE - 1), jnp.float32)],
        axis=1)


def _aux_build(mailbox, mail_time, mem_time):
    grid = (N // _RAUX,)
    return pl.pallas_call(
        _aux_body,
        grid=grid,
        in_specs=[
            pl.BlockSpec((_RAUX, D), lambda i: (i, 2)),
            pl.BlockSpec((1, 1, _RAUX), lambda i: (i, 0, 0)),
            pl.BlockSpec((1, 1, _RAUX), lambda i: (i, 0, 0)),
        ],
        out_specs=pl.BlockSpec((_RAUX, D), lambda i: (i, 0)),
        out_shape=jax.ShapeDtypeStruct((N, D), jnp.float32),
    )(mailbox, mail_time.reshape(N // _RAUX, 1, _RAUX),
      mem_time.reshape(N // _RAUX, 1, _RAUX))


def _gru_body(mlo_ref, mhi_ref, aux_ref, mem_ref, nfeat_ref,
              wt_ref, bt_ref, wx_ref, bih_ref, whh_ref, bhh_ref, ht_ref):
    f32 = jnp.float32
    bf = jnp.bfloat16
    delta = aux_ref[:, DE:DE + 1]
    te = jnp.cos(delta * wt_ref[...] + bt_ref[...])
    x = jnp.concatenate(
        [mlo_ref[...].astype(bf), mhi_ref[...].astype(bf),
         aux_ref[...].astype(bf), te.astype(bf)], axis=1)
    gx = jnp.dot(x, wx_ref[...], preferred_element_type=f32) + bih_ref[...]
    h_prev = mem_ref[...]
    gh = (jnp.dot(h_prev.astype(bf), whh_ref[...], preferred_element_type=f32)
          + bhh_ref[...])
    r = jax.nn.sigmoid(gx[:, :D] + gh[:, :D])
    z = jax.nn.sigmoid(gx[:, D:2 * D] + gh[:, D:2 * D])
    n = jnp.tanh(gx[:, 2 * D:] + r * gh[:, 2 * D:])
    new_mem = (1.0 - z) * n + z * h_prev
    ht_ref[...] = (nfeat_ref[...] + new_mem).T.astype(bf)


def _gru(mem_g, mbox_g, nfeat_g, aux_g, w_t, b_t, W_ih, b_ih, W_hh, b_hh):
    R = 1024
    grid = (E // R,)
    bf = jnp.bfloat16
    W_ihT = W_ih.T
    W_aux = jnp.zeros((D, 3 * D), jnp.float32).at[:DE].set(W_ihT[2 * D:2 * D + DE])
    full = lambda i: (0, 0)
    return pl.pallas_call(
        _gru_body,
        grid=grid,
        in_specs=[
            pl.BlockSpec((R, D), lambda i: (i, 0)),
            pl.BlockSpec((R, D), lambda i: (i, 1)),
            pl.BlockSpec((R, D), lambda i: (i, 0)),
            pl.BlockSpec((R, D), lambda i: (i, 0)),
            pl.BlockSpec((R, D), lambda i: (i, 0)),
            pl.BlockSpec((1, DT), full),
            pl.BlockSpec((1, DT), full),
            pl.BlockSpec((2 * D + D + DT, 3 * D), full),
            pl.BlockSpec((1, 3 * D), full),
            pl.BlockSpec((D, 3 * D), full),
            pl.BlockSpec((1, 3 * D), full),
        ],
        out_specs=pl.BlockSpec((D, R), lambda i: (0, i)),
        out_shape=jax.ShapeDtypeStruct((D, E), jnp.bfloat16),
    )(mbox_g, mbox_g, aux_g, mem_g, nfeat_g,
      w_t.reshape(1, DT), b_t.reshape(1, DT),
      jnp.concatenate([W_ihT[:2 * D], W_aux, W_ihT[2 * D + DE:]],
                      axis=0).astype(bf),
      b_ih.reshape(1, 3 * D), W_hh.T.astype(bf), b_hh.reshape(1, 3 * D))


_RA = 256


def _attn_body(dshT_ref, srhT_ref, dstt_ref, nbrT_ref, ef_ref,
               wt_ref, bt_ref, wqh_ref, wqt_ref,
               wkh_ref, wke_ref, wkt_ref, wvh_ref, wve_ref, wvt_ref,
               woh_ref, wo1_ref, wo2_ref, bo_ref, embT_ref):
    R = _RA
    f32 = jnp.float32
    bf = jnp.bfloat16
    wt = wt_ref[...]
    bt = bt_ref[...]
    dshT = dshT_ref[...]
    tzT = jnp.cos(bt)
    qT = (jnp.dot(wqh_ref[...], dshT, preferred_element_type=f32)
          + jnp.dot(wqt_ref[...], tzT.astype(bf), preferred_element_type=f32))
    srhT = srhT_ref[...]
    kkh = jnp.dot(wkh_ref[...], srhT, preferred_element_type=f32)
    vvh = jnp.dot(wvh_ref[...], srhT, preferred_element_type=f32)
    ef2 = ef_ref[...]
    dtT = dstt_ref[...] - nbrT_ref[...]
    a1, a2, vs = [], [], []
    for k in range(K):
        teT = jnp.cos(wt * dtT[k:k + 1, :] + bt)
        ef_k = ef2[:, k * DE:(k + 1) * DE].T.astype(bf)
        kkT = (kkh[:, k * R:(k + 1) * R]
               + jnp.dot(wke_ref[...], ef_k, preferred_element_type=f32)
               + jnp.dot(wkt_ref[...], teT.astype(bf), preferred_element_type=f32))
        vvT = (vvh[:, k * R:(k + 1) * R]
               + jnp.dot(wve_ref[...], ef_k, preferred_element_type=f32)
               + jnp.dot(wvt_ref[...], teT.astype(bf), preferred_element_type=f32))
        p = qT * kkT
        a1.append(jnp.sum(p[:DH], axis=0, keepdims=True))
        a2.append(jnp.sum(p[DH:], axis=0, keepdims=True))
        vs.append(vvT)
    scale = 1.0 / (DH ** 0.5)
    A1 = jnp.concatenate(a1, axis=0) * scale
    A2 = jnp.concatenate(a2, axis=0) * scale
    A1 = jnp.exp(A1 - jnp.max(A1, axis=0, keepdims=True))
    A2 = jnp.exp(A2 - jnp.max(A2, axis=0, keepdims=True))
    A1 = A1 / jnp.sum(A1, axis=0, keepdims=True)
    A2 = A2 / jnp.sum(A2, axis=0, keepdims=True)
    o1 = jnp.zeros((DH, R), f32)
    o2 = jnp.zeros((DH, R), f32)
    for k in range(K):
        o1 = o1 + A1[k:k + 1, :] * vs[k][:DH]
        o2 = o2 + A2[k:k + 1, :] * vs[k][DH:]
    embT = (jnp.dot(woh_ref[...], dshT, preferred_element_type=f32)
            + jnp.dot(wo1_ref[...], o1.astype(bf), preferred_element_type=f32)
            + jnp.dot(wo2_ref[...], o2.astype(bf), preferred_element_type=f32)
            + bo_ref[...])
    embT_ref[...] = jnp.maximum(embT, 0.0)


def _attn(hT, dstt2, nbrT, ef2, w_t, b_t, Wq, Wk, Wv, Wo, bo):
    grid = (B // _RA,)
    bf = jnp.bfloat16
    full = lambda i: (0, 0)
    return pl.pallas_call(
        _attn_body,
        grid=grid,
        in_specs=[
            pl.BlockSpec((D, _RA), lambda i: (0, i)),
            pl.BlockSpec((D, _RA * K), lambda i: (0, i + 1)),
            pl.BlockSpec((1, _RA), lambda i: (0, i)),
            pl.BlockSpec((K, _RA), lambda i: (0, i)),
            pl.BlockSpec((_RA, K * DE), lambda i: (i, 0)),
            pl.BlockSpec((DT, 1), full),
            pl.BlockSpec((DT, 1), full),
            pl.BlockSpec((D, D), full),
            pl.BlockSpec((D, DT), full),
            pl.BlockSpec((D, D), full),
            pl.BlockSpec((D, DE), full),
            pl.BlockSpec((D, DT), full),
            pl.BlockSpec((D, D), full),
            pl.BlockSpec((D, DE), full),
            pl.BlockSpec((D, DT), full),
            pl.BlockSpec((D, D), full),
            pl.BlockSpec((D, DH), full),
            pl.BlockSpec((D, DH), full),
            pl.BlockSpec((D, 1), full),
        ],
        out_specs=pl.BlockSpec((D, _RA), lambda i: (0, i)),
        out_shape=jax.ShapeDtypeStruct((D, B), jnp.float32),
    )(hT, hT, dstt2, nbrT, ef2,
      w_t.reshape(DT, 1), b_t.reshape(DT, 1),
      Wq[:D].T.astype(bf), Wq[D:].T.astype(bf),
      Wk[:D].T.astype(bf), Wk[D:D + DE].T.astype(bf), Wk[D + DE:].T.astype(bf),
      Wv[:D].T.astype(bf), Wv[D:D + DE].T.astype(bf), Wv[D + DE:].T.astype(bf),
      Wo[:D].T.astype(bf), Wo[D:D + DH].T.astype(bf), Wo[D + DH:].T.astype(bf),
      bo.reshape(D, 1))


def _pred_body(srcT_ref, dstT_ref, ws_ref, bs_ref, wd_ref, bd_ref, wo_ref, bo_ref,
               out_ref):
    f32 = jnp.float32
    bf = jnp.bfloat16
    hid = (jnp.dot(ws_ref[...], srcT_ref[...].astype(bf), preferred_element_type=f32)
           + jnp.dot(wd_ref[...], dstT_ref[...].astype(bf), preferred_element_type=f32)
           + bs_ref[...] + bd_ref[...])
    hid = jnp.maximum(hid, 0.0)
    out_ref[...] = (jnp.dot(wo_ref[...], hid.astype(bf), preferred_element_type=f32)
                    + bo_ref[...])


def _pred(embT, W_src, b_src, W_dst, b_dst, W_out, b_out):
    Bh = B // 2
    woT = jnp.zeros((8, D), jnp.float32).at[0:1].set(W_out.T)
    return pl.pallas_call(
        _pred_body,
        grid=(1,),
        in_specs=[
            pl.BlockSpec((D, Bh), lambda i: (0, 0)),
            pl.BlockSpec((D, Bh), lambda i: (0, 1)),
            pl.BlockSpec((D, D), lambda i: (0, 0)),
            pl.BlockSpec((D, 1), lambda i: (0, 0)),
            pl.BlockSpec((D, D), lambda i: (0, 0)),
            pl.BlockSpec((D, 1), lambda i: (0, 0)),
            pl.BlockSpec((8, D), lambda i: (0, 0)),
            pl.BlockSpec((1, 1), lambda i: (0, 0)),
        ],
        out_specs=pl.BlockSpec((8, Bh), lambda i: (0, 0)),
        out_shape=jax.ShapeDtypeStruct((8, Bh), jnp.float32),
    )(embT, embT, W_src.T.astype(jnp.bfloat16), b_src.reshape(D, 1),
      W_dst.T.astype(jnp.bfloat16), b_dst.reshape(D, 1),
      woT.astype(jnp.bfloat16), b_out.reshape(1, 1))


def kernel(dst_ids, src_ids, dst_times, nbr_times, efeat, mem, mem_time,
           mailbox, mail_time, nfeat, w_t, b_t, W_ih, b_ih, W_hh, b_hh,
           Wq, Wk, Wv, Wo, bo, W_src, b_src, W_dst, b_dst, W_out, b_out):
    nb = B // _RA
    src_km = src_ids.reshape(nb, _RA, K).transpose(0, 2, 1).reshape(-1)
    nodes = jnp.concatenate([dst_ids, src_km], axis=0).astype(jnp.int32)
    nbrT = nbr_times.reshape(B, K).T
    ef2 = efeat.reshape(B, K * DE)
    dstt2 = dst_times.reshape(1, B)
    aux = _aux_build(mailbox, mail_time, mem_time)
    mem_g, nfeat_g = _gather_mn(nodes, mem, nfeat)
    mailbox2, _o1 = lax.optimization_barrier((mailbox, mem_g[0, :8]))
    mbox_g = _gather_mb(nodes, mailbox2)
    aux2, _o2 = lax.optimization_barrier((aux, mbox_g[0, :8]))
    aux_g = _gather_ax(nodes, aux2)
    hT = _gru(mem_g, mbox_g, nfeat_g, aux_g, w_t, b_t, W_ih, b_ih, W_hh, b_hh)
    embT = _attn(hT, dstt2, nbrT, ef2, w_t, b_t, Wq, Wk, Wv, Wo, bo)
    scoreT = _pred(embT, W_src, b_src, W_dst, b_dst, W_out, b_out)
    return scoreT[0:1, :].reshape(B // 2, 1)


# speedup vs baseline: 1.1246x; 1.1246x over previous
import jax
import jax.numpy as jnp
from jax import lax
from jax.experimental import pallas as pl
from jax.experimental.pallas import tpu as pltpu
from jax.experimental.pallas import tpu_sc as plsc

N = 100000
B = 4096
K = 16
D = 128
DE = 16
DT = 100
H = 2
DH = D // H
E = B + B * K

_NC = 2
_NS = 16
_NW = _NC * _NS
_CH = 128
_PER_W = E // _NW
_NCHUNK = _PER_W // _CH


def _sc_gather_mn_body(nodes, mem, nfeat, o_mem, o_nfeat,
                       idx_all, mem_v, nfeat_v, sem):
    cid = lax.axis_index("c")
    sid = lax.axis_index("s")
    wid = sid * _NC + cid
    base = wid * _PER_W
    pltpu.sync_copy(nodes.at[pl.ds(base, _PER_W)], idx_all)

    def chunk(c, carry):
        off = base + c * _CH
        idx = idx_all.at[pl.ds(c * _CH, _CH)]
        cp1 = pltpu.async_copy(mem.at[idx], mem_v, sem)
        cp2 = pltpu.async_copy(nfeat.at[idx], nfeat_v, sem)
        cp1.wait()
        cp2.wait()
        pltpu.sync_copy(mem_v, o_mem.at[pl.ds(off, _CH)])
        pltpu.sync_copy(nfeat_v, o_nfeat.at[pl.ds(off, _CH)])
        return carry

    lax.fori_loop(0, _NCHUNK, chunk, 0)


def _sc_gather_mb_body(nodes, mbox, o_mbox, idx_all, mbox_v, sem):
    cid = lax.axis_index("c")
    sid = lax.axis_index("s")
    wid = sid * _NC + cid
    base = wid * _PER_W
    pltpu.sync_copy(nodes.at[pl.ds(base, _PER_W)], idx_all)

    def chunk(c, carry):
        off = base + c * _CH
        idx = idx_all.at[pl.ds(c * _CH, _CH)]
        pltpu.async_copy(mbox.at[idx, pl.ds(0, 2 * D)], mbox_v, sem).wait()
        pltpu.sync_copy(mbox_v, o_mbox.at[pl.ds(off, _CH)])
        return carry

    lax.fori_loop(0, _NCHUNK, chunk, 0)


def _sc_gather_ax_body(nodes, aux, o_aux, idx_all, aux_v, sem):
    cid = lax.axis_index("c")
    sid = lax.axis_index("s")
    wid = sid * _NC + cid
    base = wid * _PER_W
    pltpu.sync_copy(nodes.at[pl.ds(base, _PER_W)], idx_all)

    def chunk(c, carry):
        off = base + c * _CH
        idx = idx_all.at[pl.ds(c * _CH, _CH)]
        pltpu.async_copy(aux.at[idx], aux_v, sem).wait()
        pltpu.sync_copy(aux_v, o_aux.at[pl.ds(off, _CH)])
        return carry

    lax.fori_loop(0, _NCHUNK, chunk, 0)


def _sc_mesh():
    return plsc.VectorSubcoreMesh(core_axis_name="c", subcore_axis_name="s",
                                  num_cores=_NC)


def _gather_mn(nodes, mem, nfeat):
    f32 = jnp.float32
    run = pl.kernel(
        _sc_gather_mn_body,
        mesh=_sc_mesh(),
        out_type=[
            jax.ShapeDtypeStruct((E, D), f32),
            jax.ShapeDtypeStruct((E, D), f32),
        ],
        scratch_types=[
            pltpu.VMEM((_PER_W,), jnp.int32),
            pltpu.VMEM((_CH, D), f32),
            pltpu.VMEM((_CH, D), f32),
            pltpu.SemaphoreType.DMA,
        ],
    )
    return run(nodes, mem, nfeat)


def _gather_mb(nodes, mailbox):
    run = pl.kernel(
        _sc_gather_mb_body,
        mesh=_sc_mesh(),
        out_type=jax.ShapeDtypeStruct((E, 2 * D), jnp.float32),
        scratch_types=[
            pltpu.VMEM((_PER_W,), jnp.int32),
            pltpu.VMEM((_CH, 2 * D), jnp.float32),
            pltpu.SemaphoreType.DMA,
        ],
    )
    return run(nodes, mailbox)


def _gather_ax(nodes, aux):
    run = pl.kernel(
        _sc_gather_ax_body,
        mesh=_sc_mesh(),
        out_type=jax.ShapeDtypeStruct((E, D), jnp.float32),
        scratch_types=[
            pltpu.VMEM((_PER_W,), jnp.int32),
            pltpu.VMEM((_CH, D), jnp.float32),
            pltpu.SemaphoreType.DMA,
        ],
    )
    return run(nodes, aux)


_RAUX = 2000


def _aux_body(tail_ref, mt_ref, met_ref, aux_ref):
    R = _RAUX
    delta_row = (mt_ref[...] - met_ref[...]).reshape(1, R)
    delta_col = delta_row.T
    aux_ref[...] = jnp.concatenate(
        [tail_ref[:, :DE], delta_col, jnp.zeros((R, D - DE - 1), jnp.float32)],
        axis=1)


def _aux_build(mailbox, mail_time, mem_time):
    grid = (N // _RAUX,)
    return pl.pallas_call(
        _aux_body,
        grid=grid,
        in_specs=[
            pl.BlockSpec((_RAUX, D), lambda i: (i, 2)),
            pl.BlockSpec((1, 1, _RAUX), lambda i: (i, 0, 0)),
            pl.BlockSpec((1, 1, _RAUX), lambda i: (i, 0, 0)),
        ],
        out_specs=pl.BlockSpec((_RAUX, D), lambda i: (i, 0)),
        out_shape=jax.ShapeDtypeStruct((N, D), jnp.float32),
    )(mailbox, mail_time.reshape(N // _RAUX, 1, _RAUX),
      mem_time.reshape(N // _RAUX, 1, _RAUX))


def _gru_body(mlo_ref, mhi_ref, aux_ref, mem_ref, nfeat_ref,
              wt_ref, bt_ref, wx_ref, bih_ref, whh_ref, bhh_ref, ht_ref):
    f32 = jnp.float32
    bf = jnp.bfloat16
    delta = aux_ref[:, DE:DE + 1]
    te = jnp.cos(delta * wt_ref[...] + bt_ref[...])
    x = jnp.concatenate(
        [mlo_ref[...].astype(bf), mhi_ref[...].astype(bf),
         aux_ref[...].astype(bf), te.astype(bf)], axis=1)
    gx = jnp.dot(x, wx_ref[...], preferred_element_type=f32) + bih_ref[...]
    h_prev = mem_ref[...]
    gh = (jnp.dot(h_prev.astype(bf), whh_ref[...], preferred_element_type=f32)
          + bhh_ref[...])
    r = jax.nn.sigmoid(gx[:, :D] + gh[:, :D])
    z = jax.nn.sigmoid(gx[:, D:2 * D] + gh[:, D:2 * D])
    n = jnp.tanh(gx[:, 2 * D:] + r * gh[:, 2 * D:])
    new_mem = (1.0 - z) * n + z * h_prev
    ht_ref[...] = (nfeat_ref[...] + new_mem).T.astype(bf)


def _gru(mem_g, mbox_g, nfeat_g, aux_g, w_t, b_t, W_ih, b_ih, W_hh, b_hh):
    R = 1024
    grid = (E // R,)
    bf = jnp.bfloat16
    W_ihT = W_ih.T
    W_aux = jnp.zeros((D, 3 * D), jnp.float32).at[:DE].set(W_ihT[2 * D:2 * D + DE])
    full = lambda i: (0, 0)
    return pl.pallas_call(
        _gru_body,
        grid=grid,
        in_specs=[
            pl.BlockSpec((R, D), lambda i: (i, 0)),
            pl.BlockSpec((R, D), lambda i: (i, 1)),
            pl.BlockSpec((R, D), lambda i: (i, 0)),
            pl.BlockSpec((R, D), lambda i: (i, 0)),
            pl.BlockSpec((R, D), lambda i: (i, 0)),
            pl.BlockSpec((1, DT), full),
            pl.BlockSpec((1, DT), full),
            pl.BlockSpec((2 * D + D + DT, 3 * D), full),
            pl.BlockSpec((1, 3 * D), full),
            pl.BlockSpec((D, 3 * D), full),
            pl.BlockSpec((1, 3 * D), full),
        ],
        out_specs=pl.BlockSpec((D, R), lambda i: (0, i)),
        out_shape=jax.ShapeDtypeStruct((D, E), jnp.bfloat16),
    )(mbox_g, mbox_g, aux_g, mem_g, nfeat_g,
      w_t.reshape(1, DT), b_t.reshape(1, DT),
      jnp.concatenate([W_ihT[:2 * D], W_aux, W_ihT[2 * D + DE:]],
                      axis=0).astype(bf),
      b_ih.reshape(1, 3 * D), W_hh.T.astype(bf), b_hh.reshape(1, 3 * D))


_RA = 256


def _attn_body(dshT_ref, srhT_ref, dstt_ref, nbrT_ref, ef_ref,
               wt_ref, bt_ref, wqh_ref, wqt_ref,
               wkh_ref, wke_ref, wkt_ref, wvh_ref, wve_ref, wvt_ref,
               woh_ref, wo1_ref, wo2_ref, bo_ref, embT_ref):
    R = _RA
    f32 = jnp.float32
    bf = jnp.bfloat16
    wt = wt_ref[...]
    bt = bt_ref[...]
    dshT = dshT_ref[...]
    tzT = jnp.cos(bt)
    qT = (jnp.dot(wqh_ref[...], dshT, preferred_element_type=f32)
          + jnp.dot(wqt_ref[...], tzT.astype(bf), preferred_element_type=f32))
    srhT = srhT_ref[...]
    kkh = jnp.dot(wkh_ref[...], srhT, preferred_element_type=f32)
    vvh = jnp.dot(wvh_ref[...], srhT, preferred_element_type=f32)
    ef2 = ef_ref[...]
    dtT = dstt_ref[...] - nbrT_ref[...]
    a1, a2, vs = [], [], []
    for k in range(K):
        teT = jnp.cos(wt * dtT[k:k + 1, :] + bt)
        ef_k = ef2[:, k * DE:(k + 1) * DE].T.astype(bf)
        kkT = (kkh[:, k * R:(k + 1) * R]
               + jnp.dot(wke_ref[...], ef_k, preferred_element_type=f32)
               + jnp.dot(wkt_ref[...], teT.astype(bf), preferred_element_type=f32))
        vvT = (vvh[:, k * R:(k + 1) * R]
               + jnp.dot(wve_ref[...], ef_k, preferred_element_type=f32)
               + jnp.dot(wvt_ref[...], teT.astype(bf), preferred_element_type=f32))
        p = qT * kkT
        a1.append(jnp.sum(p[:DH], axis=0, keepdims=True))
        a2.append(jnp.sum(p[DH:], axis=0, keepdims=True))
        vs.append(vvT)
    scale = 1.0 / (DH ** 0.5)
    A1 = jnp.concatenate(a1, axis=0) * scale
    A2 = jnp.concatenate(a2, axis=0) * scale
    A1 = jnp.exp(A1 - jnp.max(A1, axis=0, keepdims=True))
    A2 = jnp.exp(A2 - jnp.max(A2, axis=0, keepdims=True))
    A1 = A1 / jnp.sum(A1, axis=0, keepdims=True)
    A2 = A2 / jnp.sum(A2, axis=0, keepdims=True)
    o1 = jnp.zeros((DH, R), f32)
    o2 = jnp.zeros((DH, R), f32)
    for k in range(K):
        o1 = o1 + A1[k:k + 1, :] * vs[k][:DH]
        o2 = o2 + A2[k:k + 1, :] * vs[k][DH:]
    embT = (jnp.dot(woh_ref[...], dshT, preferred_element_type=f32)
            + jnp.dot(wo1_ref[...], o1.astype(bf), preferred_element_type=f32)
            + jnp.dot(wo2_ref[...], o2.astype(bf), preferred_element_type=f32)
            + bo_ref[...])
    embT_ref[...] = jnp.maximum(embT, 0.0)


def _attn(hT, dstt2, nbrT, ef2, w_t, b_t, Wq, Wk, Wv, Wo, bo):
    grid = (B // _RA,)
    bf = jnp.bfloat16
    full = lambda i: (0, 0)
    return pl.pallas_call(
        _attn_body,
        grid=grid,
        in_specs=[
            pl.BlockSpec((D, _RA), lambda i: (0, i)),
            pl.BlockSpec((D, _RA * K), lambda i: (0, i + 1)),
            pl.BlockSpec((1, _RA), lambda i: (0, i)),
            pl.BlockSpec((K, _RA), lambda i: (0, i)),
            pl.BlockSpec((_RA, K * DE), lambda i: (i, 0)),
            pl.BlockSpec((DT, 1), full),
            pl.BlockSpec((DT, 1), full),
            pl.BlockSpec((D, D), full),
            pl.BlockSpec((D, DT), full),
            pl.BlockSpec((D, D), full),
            pl.BlockSpec((D, DE), full),
            pl.BlockSpec((D, DT), full),
            pl.BlockSpec((D, D), full),
            pl.BlockSpec((D, DE), full),
            pl.BlockSpec((D, DT), full),
            pl.BlockSpec((D, D), full),
            pl.BlockSpec((D, DH), full),
            pl.BlockSpec((D, DH), full),
            pl.BlockSpec((D, 1), full),
        ],
        out_specs=pl.BlockSpec((D, _RA), lambda i: (0, i)),
        out_shape=jax.ShapeDtypeStruct((D, B), jnp.float32),
    )(hT, hT, dstt2, nbrT, ef2,
      w_t.reshape(DT, 1), b_t.reshape(DT, 1),
      Wq[:D].T.astype(bf), Wq[D:].T.astype(bf),
      Wk[:D].T.astype(bf), Wk[D:D + DE].T.astype(bf), Wk[D + DE:].T.astype(bf),
      Wv[:D].T.astype(bf), Wv[D:D + DE].T.astype(bf), Wv[D + DE:].T.astype(bf),
      Wo[:D].T.astype(bf), Wo[D:D + DH].T.astype(bf), Wo[D + DH:].T.astype(bf),
      bo.reshape(D, 1))


def _pred_body(srcT_ref, dstT_ref, ws_ref, bs_ref, wd_ref, bd_ref, wo_ref, bo_ref,
               out_ref):
    f32 = jnp.float32
    bf = jnp.bfloat16
    hid = (jnp.dot(ws_ref[...], srcT_ref[...].astype(bf), preferred_element_type=f32)
           + jnp.dot(wd_ref[...], dstT_ref[...].astype(bf), preferred_element_type=f32)
           + bs_ref[...] + bd_ref[...])
    hid = jnp.maximum(hid, 0.0)
    out_ref[...] = (jnp.dot(wo_ref[...], hid.astype(bf), preferred_element_type=f32)
                    + bo_ref[...])


def _pred(embT, W_src, b_src, W_dst, b_dst, W_out, b_out):
    Bh = B // 2
    woT = jnp.zeros((8, D), jnp.float32).at[0:1].set(W_out.T)
    return pl.pallas_call(
        _pred_body,
        grid=(1,),
        in_specs=[
            pl.BlockSpec((D, Bh), lambda i: (0, 0)),
            pl.BlockSpec((D, Bh), lambda i: (0, 1)),
            pl.BlockSpec((D, D), lambda i: (0, 0)),
            pl.BlockSpec((D, 1), lambda i: (0, 0)),
            pl.BlockSpec((D, D), lambda i: (0, 0)),
            pl.BlockSpec((D, 1), lambda i: (0, 0)),
            pl.BlockSpec((8, D), lambda i: (0, 0)),
            pl.BlockSpec((1, 1), lambda i: (0, 0)),
        ],
        out_specs=pl.BlockSpec((8, Bh), lambda i: (0, 0)),
        out_shape=jax.ShapeDtypeStruct((8, Bh), jnp.float32),
    )(embT, embT, W_src.T.astype(jnp.bfloat16), b_src.reshape(D, 1),
      W_dst.T.astype(jnp.bfloat16), b_dst.reshape(D, 1),
      woT.astype(jnp.bfloat16), b_out.reshape(1, 1))


def kernel(dst_ids, src_ids, dst_times, nbr_times, efeat, mem, mem_time,
           mailbox, mail_time, nfeat, w_t, b_t, W_ih, b_ih, W_hh, b_hh,
           Wq, Wk, Wv, Wo, bo, W_src, b_src, W_dst, b_dst, W_out, b_out):
    nb = B // _RA
    src_km = src_ids.reshape(nb, _RA, K).transpose(0, 2, 1).reshape(-1)
    nodes = jnp.concatenate([dst_ids, src_km], axis=0).astype(jnp.int32)
    nbrT = nbr_times.reshape(B, K).T
    ef2 = efeat.reshape(B, K * DE)
    dstt2 = dst_times.reshape(1, B)
    aux = _aux_build(mailbox, mail_time, mem_time)
    mem_g, nfeat_g = _gather_mn(nodes, mem, nfeat)
    nodes_mb, _o1 = lax.optimization_barrier((nodes, mem_g[0, :8]))
    mbox_g = _gather_mb(nodes_mb, mailbox)
    nodes_ax, _o2 = lax.optimization_barrier((nodes, mbox_g[0, :8]))
    aux_g = _gather_ax(nodes_ax, aux)
    hT = _gru(mem_g, mbox_g, nfeat_g, aux_g, w_t, b_t, W_ih, b_ih, W_hh, b_hh)
    embT = _attn(hT, dstt2, nbrT, ef2, w_t, b_t, Wq, Wk, Wv, Wo, bo)
    scoreT = _pred(embT, W_src, b_src, W_dst, b_dst, W_out, b_out)
    return scoreT[0:1, :].reshape(B // 2, 1)
